# baseline (speedup 1.0000x reference)
import numpy as np

import jax
import jax.numpy as jnp
from jax import lax
from jax.experimental import pallas as pl
from jax.experimental.pallas import tpu as pltpu
from jax.experimental.pallas import tpu_sc as plsc
from jax._src.random.threefry2x32 import threefry2x32_p

_VOCAB = 151936
_H = 1024
_B, _S = 4, 2048
_NTOK = _B * _S

_NC, _NS = 2, 16
_NW = _NC * _NS
_TOK_PER_W = _NTOK // _NW
_CHUNK = 64
_NCHUNK = _TOK_PER_W // _CHUNK


def _sc_gather_body(ids_hbm, table_hbm, out_hbm, idx_v, rows_v, sem):
    wid = lax.axis_index("s") * _NC + lax.axis_index("c")
    base = wid * _TOK_PER_W
    for c in range(_NCHUNK):
        off = base + c * _CHUNK
        pltpu.sync_copy(ids_hbm.at[pl.ds(off, _CHUNK)], idx_v)
        pltpu.async_copy(table_hbm.at[idx_v], rows_v, sem).wait()
        pltpu.sync_copy(rows_v, out_hbm.at[pl.ds(off, _CHUNK)])


_SC_GATHER_CACHE = []


def _sc_gather(ids, table):
    if not _SC_GATHER_CACHE:
        _SC_GATHER_CACHE.append(pl.kernel(
            _sc_gather_body,
            out_type=jax.ShapeDtypeStruct((_NTOK, _H), jnp.float32),
            mesh=plsc.VectorSubcoreMesh(core_axis_name="c", subcore_axis_name="s"),
            scratch_types=[
                pltpu.VMEM((_CHUNK,), jnp.int32),
                pltpu.VMEM((_CHUNK, _H), jnp.float32),
                pltpu.SemaphoreType.DMA,
            ],
        ))
    return _SC_GATHER_CACHE[0](ids, table)


_K1 = np.uint32(0)
_K2 = np.uint32(42)
_LO = np.float32(np.nextafter(np.float32(-1.0), np.float32(0.0)))
_SPAN = np.float32(np.float32(1.0) - _LO)
_SQRT2 = np.float32(np.sqrt(2.0))
_EXP1F = np.uint32(0x3F800000)


def _noise_block(flat_base, shape):
    r = lax.broadcasted_iota(jnp.uint32, shape, 0)
    c = lax.broadcasted_iota(jnp.uint32, shape, 1)
    cnt = flat_base.astype(jnp.uint32) + r * np.uint32(shape[1]) + c
    zero = jnp.zeros(shape, jnp.uint32)
    b1, b2 = threefry2x32_p.bind(_K1, _K2, zero, cnt)
    bits = b1 ^ b2
    fb = (bits >> jnp.uint32(9)) | _EXP1F
    f = lax.bitcast_convert_type(fb, jnp.float32) - np.float32(1.0)
    u = jnp.maximum(_LO, f * _SPAN + _LO)
    return (_SQRT2 * lax.erf_inv(u)) * np.float32(0.1)


def _mm_body(x_ref, w_ref, b_ref, o_ref):
    i = pl.program_id(0)
    noise = _noise_block(i * (_MM_BLK * _H), (_MM_BLK, _H))
    acc = lax.dot_general(
        x_ref[...].astype(jnp.bfloat16), w_ref[...].astype(jnp.bfloat16),
        dimension_numbers=(((1,), (1,)), ((), ())),
        preferred_element_type=jnp.float32,
    )
    o_ref[...] = acc + b_ref[...] + noise


_MM_BLK = 512


def _linear_noise(x, W, b):
    grid = (_NTOK // _MM_BLK,)
    return pl.pallas_call(
        _mm_body,
        grid=grid,
        in_specs=[
            pl.BlockSpec((_MM_BLK, _H), lambda i: (i, 0)),
            pl.BlockSpec((_H, _H), lambda i: (0, 0)),
            pl.BlockSpec((1, _H), lambda i: (0, 0)),
        ],
        out_specs=pl.BlockSpec((_MM_BLK, _H), lambda i: (i, 0)),
        out_shape=jax.ShapeDtypeStruct((_NTOK, _H), jnp.float32),
        compiler_params=pltpu.CompilerParams(
            dimension_semantics=("parallel",),
        ),
    )(x, W, b.reshape(1, _H))


def kernel(input_ids, emb_table, W, b):
    ids = input_ids.reshape(_NTOK).astype(jnp.int32)
    emb = _sc_gather(ids, emb_table)
    out = _linear_noise(emb, W, b)
    return out.reshape(_B, _S, _H)


# speedup vs baseline: 1.1335x; 1.1335x over previous
import numpy as np

import jax
import jax.numpy as jnp
from jax import lax
from jax.experimental import pallas as pl
from jax.experimental.pallas import tpu as pltpu
from jax.experimental.pallas import tpu_sc as plsc
from jax._src.random.threefry2x32 import threefry2x32_p

_VOCAB = 151936
_H = 1024
_B, _S = 4, 2048
_NTOK = _B * _S

_NC, _NS = 2, 16
_NW = _NC * _NS
_TOK_PER_W = _NTOK // _NW
_CHUNK = 64
_NCHUNK = _TOK_PER_W // _CHUNK


def _sc_gather_body(ids_hbm, table_hbm, out_hbm, idx_v, rows_v, sem):
    wid = lax.axis_index("s") * _NC + lax.axis_index("c")
    base = wid * _TOK_PER_W
    for c in range(_NCHUNK):
        off = base + c * _CHUNK
        pltpu.sync_copy(ids_hbm.at[pl.ds(off, _CHUNK)], idx_v)
        pltpu.async_copy(table_hbm.at[idx_v], rows_v, sem).wait()
        pltpu.sync_copy(rows_v, out_hbm.at[pl.ds(off, _CHUNK)])


_SC_GATHER_CACHE = []


def _sc_gather(ids, table):
    if not _SC_GATHER_CACHE:
        _SC_GATHER_CACHE.append(pl.kernel(
            _sc_gather_body,
            out_type=jax.ShapeDtypeStruct((_NTOK, _H), jnp.float32),
            mesh=plsc.VectorSubcoreMesh(core_axis_name="c", subcore_axis_name="s"),
            scratch_types=[
                pltpu.VMEM((_CHUNK,), jnp.int32),
                pltpu.VMEM((_CHUNK, _H), jnp.float32),
                pltpu.SemaphoreType.DMA,
            ],
        ))
    return _SC_GATHER_CACHE[0](ids, table)


_K1 = np.uint32(0)
_K2 = np.uint32(42)
_LO = np.float32(np.nextafter(np.float32(-1.0), np.float32(0.0)))
_SPAN = np.float32(np.float32(1.0) - _LO)
_SQRT2 = np.float32(np.sqrt(2.0))
_EXP1F = np.uint32(0x3F800000)


_Q = tuple(np.float32(v) for v in (
    1.2531494, 0.0021799551, 0.32501802, -0.012888024,
    0.050362255, -0.026957497, 0.0036111893))


def _noise_block(flat_base, shape):
    r = lax.broadcasted_iota(jnp.uint32, shape, 0)
    c = lax.broadcasted_iota(jnp.uint32, shape, 1)
    cnt = flat_base.astype(jnp.uint32) + r * np.uint32(shape[1]) + c
    zero = jnp.zeros(shape, jnp.uint32)
    b1, b2 = threefry2x32_p.bind(_K1, _K2, zero, cnt)
    bits = b1 ^ b2
    fb = (bits >> jnp.uint32(9)) | _EXP1F
    f = lax.bitcast_convert_type(fb, jnp.float32) - np.float32(1.0)
    u = jnp.maximum(_LO, f * _SPAN + _LO)
    w = -jnp.log(np.float32(1.0) - u * u)
    s = jnp.sqrt(w)
    q = jnp.full(shape, _Q[6], jnp.float32)
    for k in (5, 4, 3, 2, 1, 0):
        q = q * s + _Q[k]
    return (u * q) * np.float32(0.1)


def _mm_body(x_ref, w_ref, b_ref, o_ref):
    i = pl.program_id(0)
    noise = _noise_block(i * (_MM_BLK * _H), (_MM_BLK, _H))
    acc = lax.dot_general(
        x_ref[...].astype(jnp.bfloat16), w_ref[...].astype(jnp.bfloat16),
        dimension_numbers=(((1,), (1,)), ((), ())),
        preferred_element_type=jnp.float32,
    )
    o_ref[...] = acc + b_ref[...] + noise


_MM_BLK = 512


def _linear_noise(x, W, b):
    grid = (_NTOK // _MM_BLK,)
    return pl.pallas_call(
        _mm_body,
        grid=grid,
        in_specs=[
            pl.BlockSpec((_MM_BLK, _H), lambda i: (i, 0)),
            pl.BlockSpec((_H, _H), lambda i: (0, 0)),
            pl.BlockSpec((1, _H), lambda i: (0, 0)),
        ],
        out_specs=pl.BlockSpec((_MM_BLK, _H), lambda i: (i, 0)),
        out_shape=jax.ShapeDtypeStruct((_NTOK, _H), jnp.float32),
        compiler_params=pltpu.CompilerParams(
            dimension_semantics=("parallel",),
        ),
    )(x, W, b.reshape(1, _H))


def kernel(input_ids, emb_table, W, b):
    ids = input_ids.reshape(_NTOK).astype(jnp.int32)
    emb = _sc_gather(ids, emb_table)
    out = _linear_noise(emb, W, b)
    return out.reshape(_B, _S, _H)


# speedup vs baseline: 1.1406x; 1.0063x over previous
import numpy as np

import jax
import jax.numpy as jnp
from jax import lax
from jax.experimental import pallas as pl
from jax.experimental.pallas import tpu as pltpu
from jax.experimental.pallas import tpu_sc as plsc
from jax._src.random.threefry2x32 import threefry2x32_p

_VOCAB = 151936
_H = 1024
_B, _S = 4, 2048
_NTOK = _B * _S

_NC, _NS = 2, 16
_NW = _NC * _NS
_TOK_PER_W = _NTOK // _NW
_CHUNK = 32
_NCHUNK = _TOK_PER_W // _CHUNK


def _sc_gather_body(ids_hbm, table_hbm, out_hbm, idx_v, rows0, rows1,
                    gsem0, gsem1, ssem0, ssem1):
    wid = lax.axis_index("s") * _NC + lax.axis_index("c")
    base = wid * _TOK_PER_W
    pltpu.sync_copy(ids_hbm.at[pl.ds(base, _TOK_PER_W)], idx_v)
    rows = (rows0, rows1)
    gsem = (gsem0, gsem1)
    ssem = (ssem0, ssem1)
    gcp = [None, None]
    scp = [None, None]

    def start(c):
        p = c & 1
        gcp[p] = pltpu.async_copy(
            table_hbm.at[idx_v.at[pl.ds(c * _CHUNK, _CHUNK)]], rows[p], gsem[p])

    start(0)
    for c in range(1, _NCHUNK):
        p = c & 1
        if scp[p] is not None:
            scp[p].wait()
        start(c)
        q = 1 - p
        gcp[q].wait()
        scp[q] = pltpu.async_copy(
            rows[q], out_hbm.at[pl.ds(base + (c - 1) * _CHUNK, _CHUNK)], ssem[q])
    p = (_NCHUNK - 1) & 1
    gcp[p].wait()
    pltpu.sync_copy(rows[p], out_hbm.at[pl.ds(base + (_NCHUNK - 1) * _CHUNK, _CHUNK)])
    if scp[1 - p] is not None:
        scp[1 - p].wait()


_SC_GATHER_CACHE = []


def _sc_gather(ids, table):
    if not _SC_GATHER_CACHE:
        _SC_GATHER_CACHE.append(pl.kernel(
            _sc_gather_body,
            out_type=jax.ShapeDtypeStruct((_NTOK, _H), jnp.float32),
            mesh=plsc.VectorSubcoreMesh(core_axis_name="c", subcore_axis_name="s"),
            scratch_types=[
                pltpu.VMEM((_TOK_PER_W,), jnp.int32),
                pltpu.VMEM((_CHUNK, _H), jnp.float32),
                pltpu.VMEM((_CHUNK, _H), jnp.float32),
                pltpu.SemaphoreType.DMA,
                pltpu.SemaphoreType.DMA,
                pltpu.SemaphoreType.DMA,
                pltpu.SemaphoreType.DMA,
            ],
        ))
    return _SC_GATHER_CACHE[0](ids, table)


_K1 = np.uint32(0)
_K2 = np.uint32(42)
_LO = np.float32(np.nextafter(np.float32(-1.0), np.float32(0.0)))
_SPAN = np.float32(np.float32(1.0) - _LO)
_SQRT2 = np.float32(np.sqrt(2.0))
_EXP1F = np.uint32(0x3F800000)


_Q = tuple(np.float32(v) for v in (
    1.2531494, 0.0021799551, 0.32501802, -0.012888024,
    0.050362255, -0.026957497, 0.0036111893))


def _noise_block(flat_base, shape):
    r = lax.broadcasted_iota(jnp.uint32, shape, 0)
    c = lax.broadcasted_iota(jnp.uint32, shape, 1)
    cnt = flat_base.astype(jnp.uint32) + r * np.uint32(shape[1]) + c
    zero = jnp.zeros(shape, jnp.uint32)
    b1, b2 = threefry2x32_p.bind(_K1, _K2, zero, cnt)
    bits = b1 ^ b2
    fb = (bits >> jnp.uint32(9)) | _EXP1F
    f = lax.bitcast_convert_type(fb, jnp.float32) - np.float32(1.0)
    u = jnp.maximum(_LO, f * _SPAN + _LO)
    w = -jnp.log(np.float32(1.0) - u * u)
    s = jnp.sqrt(w)
    q = jnp.full(shape, _Q[6], jnp.float32)
    for k in (5, 4, 3, 2, 1, 0):
        q = q * s + _Q[k]
    return (u * q) * np.float32(0.1)


def _mm_body(x_ref, w_ref, b_ref, o_ref):
    i = pl.program_id(0)
    noise = _noise_block(i * (_MM_BLK * _H), (_MM_BLK, _H))
    acc = lax.dot_general(
        x_ref[...].astype(jnp.bfloat16), w_ref[...].astype(jnp.bfloat16),
        dimension_numbers=(((1,), (1,)), ((), ())),
        preferred_element_type=jnp.float32,
    )
    o_ref[...] = acc + b_ref[...] + noise


_MM_BLK = 512


def _linear_noise(x, W, b):
    grid = (_NTOK // _MM_BLK,)
    return pl.pallas_call(
        _mm_body,
        grid=grid,
        in_specs=[
            pl.BlockSpec((_MM_BLK, _H), lambda i: (i, 0)),
            pl.BlockSpec((_H, _H), lambda i: (0, 0)),
            pl.BlockSpec((1, _H), lambda i: (0, 0)),
        ],
        out_specs=pl.BlockSpec((_MM_BLK, _H), lambda i: (i, 0)),
        out_shape=jax.ShapeDtypeStruct((_NTOK, _H), jnp.float32),
        compiler_params=pltpu.CompilerParams(
            dimension_semantics=("parallel",),
        ),
    )(x, W, b.reshape(1, _H))


def kernel(input_ids, emb_table, W, b):
    ids = input_ids.reshape(_NTOK).astype(jnp.int32)
    emb = _sc_gather(ids, emb_table)
    out = _linear_noise(emb, W, b)
    return out.reshape(_B, _S, _H)


# speedup vs baseline: 1.1785x; 1.0332x over previous
import numpy as np

import jax
import jax.numpy as jnp
from jax import lax
from jax.experimental import pallas as pl
from jax.experimental.pallas import tpu as pltpu
from jax.experimental.pallas import tpu_sc as plsc
from jax._src.random.threefry2x32 import threefry2x32_p

_VOCAB = 151936
_H = 1024
_B, _S = 4, 2048
_NTOK = _B * _S

_NC, _NS = 2, 16
_NW = _NC * _NS
_TOK_PER_W = _NTOK // _NW
_CHUNK = 32
_NCHUNK = _TOK_PER_W // _CHUNK


def _sc_gather_body(ids_hbm, table_hbm, out_hbm, idx_v, rows0, rows1,
                    gsem0, gsem1, ssem0, ssem1, *, tok_per_w=_TOK_PER_W,
                    nchunk=_NCHUNK):
    wid = lax.axis_index("s") * _NC + lax.axis_index("c")
    base = wid * tok_per_w
    pltpu.sync_copy(ids_hbm.at[pl.ds(base, tok_per_w)], idx_v)
    rows = (rows0, rows1)
    gsem = (gsem0, gsem1)
    ssem = (ssem0, ssem1)
    gcp = [None, None]
    scp = [None, None]

    def start(c):
        p = c & 1
        gcp[p] = pltpu.async_copy(
            table_hbm.at[idx_v.at[pl.ds(c * _CHUNK, _CHUNK)]], rows[p], gsem[p])

    start(0)
    for c in range(1, nchunk):
        p = c & 1
        if scp[p] is not None:
            scp[p].wait()
        start(c)
        q = 1 - p
        gcp[q].wait()
        scp[q] = pltpu.async_copy(
            rows[q], out_hbm.at[pl.ds(base + (c - 1) * _CHUNK, _CHUNK)], ssem[q])
    p = (nchunk - 1) & 1
    gcp[p].wait()
    pltpu.sync_copy(rows[p], out_hbm.at[pl.ds(base + (nchunk - 1) * _CHUNK, _CHUNK)])
    if scp[1 - p] is not None:
        scp[1 - p].wait()


_SC_GATHER_CACHE = {}


def _sc_gather(ids, table):
    n = ids.shape[0]
    if n not in _SC_GATHER_CACHE:
        tok_per_w = n // _NW
        nchunk = tok_per_w // _CHUNK
        _SC_GATHER_CACHE[n] = pl.kernel(
            lambda *a: _sc_gather_body(*a, tok_per_w=tok_per_w, nchunk=nchunk),
            out_type=jax.ShapeDtypeStruct((n, _H), jnp.float32),
            mesh=plsc.VectorSubcoreMesh(core_axis_name="c", subcore_axis_name="s"),
            scratch_types=[
                pltpu.VMEM((tok_per_w,), jnp.int32),
                pltpu.VMEM((_CHUNK, _H), jnp.float32),
                pltpu.VMEM((_CHUNK, _H), jnp.float32),
                pltpu.SemaphoreType.DMA,
                pltpu.SemaphoreType.DMA,
                pltpu.SemaphoreType.DMA,
                pltpu.SemaphoreType.DMA,
            ],
        )
    return _SC_GATHER_CACHE[n](ids, table)


_K1 = np.uint32(0)
_K2 = np.uint32(42)
_LO = np.float32(np.nextafter(np.float32(-1.0), np.float32(0.0)))
_SPAN = np.float32(np.float32(1.0) - _LO)
_SQRT2 = np.float32(np.sqrt(2.0))
_EXP1F = np.uint32(0x3F800000)


_Q = tuple(np.float32(v) for v in (
    1.2531494, 0.0021799551, 0.32501802, -0.012888024,
    0.050362255, -0.026957497, 0.0036111893))


def _noise_block(flat_base, shape):
    r = lax.broadcasted_iota(jnp.uint32, shape, 0)
    c = lax.broadcasted_iota(jnp.uint32, shape, 1)
    cnt = flat_base.astype(jnp.uint32) + r * np.uint32(shape[1]) + c
    zero = jnp.zeros(shape, jnp.uint32)
    b1, b2 = threefry2x32_p.bind(_K1, _K2, zero, cnt)
    bits = b1 ^ b2
    fb = (bits >> jnp.uint32(9)) | _EXP1F
    f = lax.bitcast_convert_type(fb, jnp.float32) - np.float32(1.0)
    u = jnp.maximum(_LO, f * _SPAN + _LO)
    w = -jnp.log(np.float32(1.0) - u * u)
    s = jnp.sqrt(w)
    q = jnp.full(shape, _Q[6], jnp.float32)
    for k in (5, 4, 3, 2, 1, 0):
        q = q * s + _Q[k]
    return (u * q) * np.float32(0.1)


_MM_BLK = 512
_HALF_BLKS = (_NTOK // _MM_BLK) // 2


def _mm_compute(x_ref, w_ref, b_ref, o_ref, base_blk):
    i = pl.program_id(0)
    noise = _noise_block((i + base_blk) * (_MM_BLK * _H), (_MM_BLK, _H))
    acc = lax.dot_general(
        x_ref[...].astype(jnp.bfloat16), w_ref[...].astype(jnp.bfloat16),
        dimension_numbers=(((1,), (1,)), ((), ())),
        preferred_element_type=jnp.float32,
    )
    o_ref[...] = acc + b_ref[...] + noise


def _mm_body_lo(x_ref, w_ref, b_ref, o_ref):
    _mm_compute(x_ref, w_ref, b_ref, o_ref, 0)


def _mm_body_hi(buf_ref, x_ref, w_ref, b_ref, o_ref):
    del buf_ref
    _mm_compute(x_ref, w_ref, b_ref, o_ref, _HALF_BLKS)


def _linear_noise_lo(x, W, b):
    return pl.pallas_call(
        _mm_body_lo,
        grid=(_HALF_BLKS,),
        in_specs=[
            pl.BlockSpec((_MM_BLK, _H), lambda i: (i, 0)),
            pl.BlockSpec((_H, _H), lambda i: (0, 0)),
            pl.BlockSpec((1, _H), lambda i: (0, 0)),
        ],
        out_specs=pl.BlockSpec((_MM_BLK, _H), lambda i: (i, 0)),
        out_shape=jax.ShapeDtypeStruct((_NTOK, _H), jnp.float32),
    )(x, W, b.reshape(1, _H))


def _linear_noise_hi(buf, x, W, b):
    return pl.pallas_call(
        _mm_body_hi,
        grid=(_HALF_BLKS,),
        in_specs=[
            pl.BlockSpec(memory_space=pltpu.HBM),
            pl.BlockSpec((_MM_BLK, _H), lambda i: (i, 0)),
            pl.BlockSpec((_H, _H), lambda i: (0, 0)),
            pl.BlockSpec((1, _H), lambda i: (0, 0)),
        ],
        out_specs=pl.BlockSpec((_MM_BLK, _H), lambda i: (i + _HALF_BLKS, 0)),
        out_shape=jax.ShapeDtypeStruct((_NTOK, _H), jnp.float32),
        input_output_aliases={0: 0},
    )(buf, x, W, b.reshape(1, _H))


def kernel(input_ids, emb_table, W, b):
    ids = input_ids.reshape(_NTOK).astype(jnp.int32)
    half = _NTOK // 2
    emb0 = _sc_gather(ids[:half], emb_table)
    emb1 = _sc_gather(ids[half:], emb_table)
    buf = _linear_noise_lo(emb0, W, b)
    out = _linear_noise_hi(buf, emb1, W, b)
    return out.reshape(_B, _S, _H)
